# baseline (speedup 1.0000x reference)
import functools

import jax
import jax.numpy as jnp
from jax import lax
from jax.experimental import pallas as pl
from jax.experimental.pallas import tpu as pltpu
from jax.experimental.pallas import tpu_sc as plsc

_B, _C, _P = 1024, 100000, 50
_PPAD = 64
_NW = 32
_CHUNK = 128
_NCH = _B * _PPAD // _NW // _CHUNK
_R = 16


def _sc_gather_body(flat_in, idx_hbm, out_hbm, idx_v, val_v, sem):
    wid = lax.axis_index("s") * 2 + lax.axis_index("c")
    pltpu.sync_copy(idx_hbm.at[wid], idx_v)
    copies = [
        pltpu.async_copy(flat_in.at[idx_v.at[j]], val_v.at[j], sem)
        for j in range(_NCH)
    ]
    for c in copies:
        c.wait()
    pltpu.sync_copy(val_v, out_hbm.at[wid])


@functools.cache
def _sc_gather():
    return pl.kernel(
        _sc_gather_body,
        mesh=plsc.VectorSubcoreMesh(core_axis_name="c", subcore_axis_name="s"),
        out_type=jax.ShapeDtypeStruct((_NW, _NCH, _CHUNK), jnp.float32),
        scratch_types=[
            pltpu.VMEM((_NCH, _CHUNK), jnp.int32),
            pltpu.VMEM((_NCH, _CHUNK), jnp.float32),
            pltpu.SemaphoreType.DMA,
        ],
    )


def _loss_body(x_ref, g_ref, o_ref):
    x = x_ref[...]
    m_all = jnp.maximum(jnp.max(x, axis=1), 0.0)
    s_all = jnp.sum(jnp.exp(x - m_all[:, None]), axis=1) + jnp.exp(-m_all)
    all_loss = m_all + jnp.log(s_all)

    g = g_ref[...]
    valid = lax.broadcasted_iota(jnp.int32, (_R, _PPAD), 1) < _P
    gm = jnp.where(valid, g, -jnp.inf)
    m_p = jnp.max(gm, axis=1)
    s_p = jnp.sum(jnp.where(valid, jnp.exp(g - m_p[:, None]), 0.0), axis=1)
    lse_pos = m_p + jnp.log(s_p)

    z = jnp.where(valid, -g, -jnp.inf)
    m_n = jnp.maximum(jnp.max(z, axis=1), 0.0)
    s_n = jnp.sum(jnp.where(valid, jnp.exp(-g - m_n[:, None]), 0.0), axis=1)
    pos_loss = m_n + jnp.log(s_n + jnp.exp(-m_n))

    aux = jnp.clip(1.0 - jnp.exp(lse_pos - all_loss), 1e-12, 1.0)
    o_ref[0, 0, :] = pos_loss + all_loss + jnp.log(aux)


_loss_call = pl.pallas_call(
    _loss_body,
    grid=(_B // _R,),
    in_specs=[
        pl.BlockSpec((_R, _C), lambda i: (i, 0)),
        pl.BlockSpec((_R, _PPAD), lambda i: (i, 0)),
    ],
    out_specs=pl.BlockSpec((1, 1, _R), lambda i: (i, 0, 0)),
    out_shape=jax.ShapeDtypeStruct((_B // _R, 1, _R), jnp.float32),
)


def kernel(input, target):
    tgt = jnp.concatenate([target, target[:, : _PPAD - _P]], axis=1)
    flat_idx = tgt + (jnp.arange(_B, dtype=jnp.int32) * _C)[:, None]
    gathered = _sc_gather()(
        input.reshape(-1), flat_idx.reshape(_NW, _NCH, _CHUNK)
    )
    out = _loss_call(input, gathered.reshape(_B, _PPAD))
    return out.reshape(_B)


# speedup vs baseline: 1.0251x; 1.0251x over previous
import functools

import jax
import jax.numpy as jnp
from jax import lax
from jax.experimental import pallas as pl
from jax.experimental.pallas import tpu as pltpu
from jax.experimental.pallas import tpu_sc as plsc

_B, _C, _P = 1024, 100000, 50
_PPAD = 64
_NW = 32
_CHUNK = 128
_NCH = _B * _PPAD // _NW // _CHUNK
_R = 16


def _sc_gather_body(flat_in, idx_hbm, out_hbm, idx_v, val_v, sem):
    wid = lax.axis_index("s") * 2 + lax.axis_index("c")
    pltpu.sync_copy(idx_hbm.at[wid], idx_v)
    copies = [
        pltpu.async_copy(flat_in.at[idx_v.at[j]], val_v.at[j], sem)
        for j in range(_NCH)
    ]
    for c in copies:
        c.wait()
    pltpu.sync_copy(val_v, out_hbm.at[wid])


@functools.cache
def _sc_gather():
    return pl.kernel(
        _sc_gather_body,
        mesh=plsc.VectorSubcoreMesh(core_axis_name="c", subcore_axis_name="s"),
        out_type=jax.ShapeDtypeStruct((_NW, _NCH, _CHUNK), jnp.float32),
        scratch_types=[
            pltpu.VMEM((_NCH, _CHUNK), jnp.int32),
            pltpu.VMEM((_NCH, _CHUNK), jnp.float32),
            pltpu.SemaphoreType.DMA,
        ],
    )


_NX = 8
_CB = 12544
_CLAST = _C - (_NX - 1) * _CB


def _loss_body(*refs):
    x_refs, g_ref, o_ref = refs[:_NX], refs[_NX], refs[_NX + 1]

    def masked(j, x):
        if j < _NX - 1:
            return x
        tail = lax.broadcasted_iota(jnp.int32, (_R, _CB), 1) < _CLAST
        return jnp.where(tail, x, -jnp.inf)

    m = jnp.full((_R,), 0.0, dtype=jnp.float32)
    for j, xr in enumerate(x_refs):
        m = jnp.maximum(m, jnp.max(masked(j, xr[...]), axis=1))
    m_all = m
    s = jnp.exp(-m_all)
    for j, xr in enumerate(x_refs):
        s = s + jnp.sum(jnp.exp(masked(j, xr[...]) - m_all[:, None]), axis=1)
    all_loss = m_all + jnp.log(s)

    g = g_ref[...]
    valid = lax.broadcasted_iota(jnp.int32, (_R, _PPAD), 1) < _P
    gm = jnp.where(valid, g, -jnp.inf)
    m_p = jnp.max(gm, axis=1)
    s_p = jnp.sum(jnp.where(valid, jnp.exp(g - m_p[:, None]), 0.0), axis=1)
    lse_pos = m_p + jnp.log(s_p)

    z = jnp.where(valid, -g, -jnp.inf)
    m_n = jnp.maximum(jnp.max(z, axis=1), 0.0)
    s_n = jnp.sum(jnp.where(valid, jnp.exp(-g - m_n[:, None]), 0.0), axis=1)
    pos_loss = m_n + jnp.log(s_n + jnp.exp(-m_n))

    aux = jnp.clip(1.0 - jnp.exp(lse_pos - all_loss), 1e-12, 1.0)
    o_ref[0, 0, :] = pos_loss + all_loss + jnp.log(aux)


_loss_call = pl.pallas_call(
    _loss_body,
    grid=(_B // _R,),
    in_specs=[
        pl.BlockSpec((_R, _CB), functools.partial(lambda j, i: (i, j), j))
        for j in range(_NX)
    ]
    + [pl.BlockSpec((_R, _PPAD), lambda i: (i, 0))],
    out_specs=pl.BlockSpec((1, 1, _R), lambda i: (i, 0, 0)),
    out_shape=jax.ShapeDtypeStruct((_B // _R, 1, _R), jnp.float32),
)


def kernel(input, target):
    tgt = jnp.concatenate([target, target[:, : _PPAD - _P]], axis=1)
    flat_idx = tgt + (jnp.arange(_B, dtype=jnp.int32) * _C)[:, None]
    gathered = _sc_gather()(
        input.reshape(-1), flat_idx.reshape(_NW, _NCH, _CHUNK)
    )
    out = _loss_call(*([input] * _NX), gathered.reshape(_B, _PPAD))
    return out.reshape(_B)


# speedup vs baseline: 1.8331x; 1.7882x over previous
import functools

import jax
import jax.numpy as jnp
from jax import lax
from jax.experimental import pallas as pl

_B, _C, _P = 1024, 100000, 50
_PPAD = 64
_R = 16
_NX = 8
_CB = 12544
_Q = _CB // 128
_CLAST = _C - (_NX - 1) * _CB


def _loss_body(*refs):
    x_refs = refs[:_NX]
    tgrp_ref, tmod_ref, o_ref = refs[_NX], refs[_NX + 1], refs[_NX + 2]

    def masked(j, x):
        if j < _NX - 1:
            return x
        tail = lax.broadcasted_iota(jnp.int32, (_R, _CB), 1) < _CLAST
        return jnp.where(tail, x, -jnp.inf)

    m = jnp.full((_R, 1), 0.0, dtype=jnp.float32)
    for j, xr in enumerate(x_refs):
        m = jnp.maximum(m, jnp.max(masked(j, xr[...]), axis=1, keepdims=True))
    s = jnp.exp(-m)
    for j, xr in enumerate(x_refs):
        s = s + jnp.sum(jnp.exp(masked(j, xr[...]) - m), axis=1, keepdims=True)
    all_loss = m + jnp.log(s)

    tgrp = tgrp_ref[...]
    q_iota = lax.broadcasted_iota(jnp.int32, (_R, _PPAD, _Q), 2)
    z = jnp.zeros((_R, _PPAD, 128), dtype=jnp.float32)
    for j, xr in enumerate(x_refs):
        sel = (tgrp[..., None] - _Q * j == q_iota).astype(jnp.float32)
        xv = xr[...]
        if j == _NX - 1:
            tail = lax.broadcasted_iota(jnp.int32, (_R, _CB), 1) < _CLAST
            xv = jnp.where(tail, xv, 0.0)
        x3 = xv.reshape(_R, _Q, 128)
        z = z + lax.dot_general(
            sel, x3, (((2,), (1,)), ((0,), (0,))),
            preferred_element_type=jnp.float32,
        )
    lane = lax.broadcasted_iota(jnp.int32, (_R, _PPAD, 128), 2)
    tmod = tmod_ref[...]
    g = jnp.sum(jnp.where(lane == tmod[..., None], z, 0.0), axis=2)

    valid = lax.broadcasted_iota(jnp.int32, (_R, _PPAD), 1) < _P
    gmask = jnp.where(valid, g, -jnp.inf)
    m_p = jnp.max(gmask, axis=1, keepdims=True)
    s_p = jnp.sum(jnp.where(valid, jnp.exp(g - m_p), 0.0), axis=1, keepdims=True)
    lse_pos = m_p + jnp.log(s_p)

    zneg = jnp.where(valid, -g, -jnp.inf)
    m_n = jnp.maximum(jnp.max(zneg, axis=1, keepdims=True), 0.0)
    s_n = jnp.sum(jnp.where(valid, jnp.exp(-g - m_n), 0.0), axis=1, keepdims=True)
    pos_loss = m_n + jnp.log(s_n + jnp.exp(-m_n))

    aux = jnp.clip(1.0 - jnp.exp(lse_pos - all_loss), 1e-12, 1.0)
    o_ref[...] = pos_loss + all_loss + jnp.log(aux)


_loss_call = pl.pallas_call(
    _loss_body,
    grid=(_B // _R,),
    in_specs=[
        pl.BlockSpec((_R, _CB), functools.partial(lambda j, i: (i, j), j))
        for j in range(_NX)
    ]
    + [
        pl.BlockSpec((_R, _PPAD), lambda i: (i, 0)),
        pl.BlockSpec((_R, _PPAD), lambda i: (i, 0)),
    ],
    out_specs=pl.BlockSpec((_R, 1), lambda i: (i, 0)),
    out_shape=jax.ShapeDtypeStruct((_B, 1), jnp.float32),
)


def kernel(input, target):
    tgt = jnp.concatenate([target, target[:, : _PPAD - _P]], axis=1)
    out = _loss_call(*([input] * _NX), tgt // 128, tgt % 128)
    return out.reshape(_B)
